# Initial kernel scaffold; baseline (speedup 1.0000x reference)
import jax
import jax.numpy as jnp
from jax.experimental import pallas as pl


def kernel(is_object, boxes, gt_boxes, obj_ids, track_obj_idx, track_gt_idx):
    raise NotImplementedError("write your pallas kernel here")


# speedup vs baseline: 18363.0772x; 18363.0772x over previous
import jax
import jax.numpy as jnp
from jax import lax
from jax.experimental import pallas as pl
from jax.experimental.pallas import tpu as pltpu


def _match_kernel(x_ref, y_ref, gx_ref, gy_ref, obj_ref, oid_ref, tobj_ref,
                  tgt_ref, gt_out, obj_out, score_out, rowfree_ref,
                  colfree_ref):
    M, N = gx_ref.shape[0], x_ref.shape[1]
    BIGI = jnp.int32(0x3FFFFFFF)
    INF = jnp.float32(jnp.inf)

    x = x_ref[...]
    y = y_ref[...]
    gx = gx_ref[...]
    gy = gy_ref[...]
    tobj = tobj_ref[...]
    tgt = tgt_ref[...]
    oid = oid_ref[...]

    iota_m = lax.broadcasted_iota(jnp.int32, (M, N), 0)
    iota_n = lax.broadcasted_iota(jnp.int32, (M, N), 1)

    match = tobj == oid
    has_match = jnp.any(match, axis=0, keepdims=True)
    jj_first = jnp.min(jnp.where(match, iota_m, BIGI), axis=0, keepdims=True)
    gt_out[...] = jnp.where(has_match, jj_first, -1).astype(jnp.int32)
    obj_out[...] = tobj
    colfree_ref[...] = (~jnp.any(match, axis=1, keepdims=True)).astype(jnp.int32)
    rowfree_ref[...] = (~((tgt >= 0) | has_match)).astype(jnp.int32)

    score_out[...] = jax.nn.sigmoid(obj_ref[...])

    def body(_):
        rowfree = rowfree_ref[...] != 0
        colfree = colfree_ref[...] != 0
        dist = (x - gx) ** 2 + (y - gy) ** 2
        dc = jnp.where(colfree, dist, INF)
        rowmin = jnp.min(dc, axis=0, keepdims=True)
        rowarg = jnp.min(jnp.where(colfree & (dist == rowmin), iota_m, BIGI),
                         axis=0, keepdims=True)
        dr = jnp.where(rowfree, dist, INF)
        colmin = jnp.min(dr, axis=1, keepdims=True)
        colarg = jnp.min(jnp.where(rowfree & (dist == colmin), iota_n, BIGI),
                         axis=1, keepdims=True)
        mutual = (iota_m == rowarg) & (iota_n == colarg)
        won_row = jnp.any(mutual, axis=0, keepdims=True)
        won_col = jnp.any(mutual, axis=1, keepdims=True)
        objval = jnp.min(jnp.where(mutual, oid, BIGI), axis=0, keepdims=True)
        gt_out[...] = jnp.where(won_row, rowarg, gt_out[...])
        obj_out[...] = jnp.where(won_row, objval, obj_out[...])
        rowfree_ref[...] = (rowfree & ~won_row).astype(jnp.int32)
        colfree_ref[...] = (colfree & ~won_col).astype(jnp.int32)
        return jnp.max(won_row.astype(jnp.int32))

    lax.while_loop(lambda c: c > 0, body, jnp.int32(1))


def kernel(is_object, boxes, gt_boxes, obj_ids, track_obj_idx, track_gt_idx):
    N = track_obj_idx.shape[0]
    M = obj_ids.shape[0]
    x = boxes[-1, 0, :, 0].reshape(1, N)
    y = boxes[-1, 0, :, 1].reshape(1, N)
    gx = gt_boxes[:, 0].reshape(M, 1)
    gy = gt_boxes[:, 1].reshape(M, 1)
    obj = is_object[-1, 0, :, 0].reshape(1, N)
    oid = obj_ids.reshape(M, 1)
    tobj = track_obj_idx.reshape(1, N)
    tgt = track_gt_idx.reshape(1, N)

    gt, ob, score = pl.pallas_call(
        _match_kernel,
        out_shape=(
            jax.ShapeDtypeStruct((1, N), jnp.int32),
            jax.ShapeDtypeStruct((1, N), jnp.int32),
            jax.ShapeDtypeStruct((1, N), jnp.float32),
        ),
        scratch_shapes=[
            pltpu.VMEM((1, N), jnp.int32),
            pltpu.VMEM((M, 1), jnp.int32),
        ],
    )(x, y, gx, gy, obj, oid, tobj, tgt)
    return gt.reshape(N), ob.reshape(N), score.reshape(N)
